# Initial kernel scaffold; baseline (speedup 1.0000x reference)
import jax
import jax.numpy as jnp
from jax.experimental import pallas as pl


def kernel(x, Wg, W1, W2, W3, Ws1, Ws2, Ws3, lres_scale):
    raise NotImplementedError("write your pallas kernel here")


# speedup vs baseline: 1.1445x; 1.1445x over previous
import functools

import jax
import jax.numpy as jnp
from jax.experimental import pallas as pl
from jax.experimental.pallas import tpu as pltpu

C = 1.0
DIM = 2048
INTER = 1408
E = 8
TOPK = 2
TOK = 2048
RB = 128


def _gate_kernel(x_ref, wg_ref, wf_ref):
    logits = jax.lax.dot_general(
        x_ref[...], wg_ref[...], (((1,), (1,)), ((), ())),
        preferred_element_type=jnp.float32,
        precision=jax.lax.Precision.DEFAULT)
    m = jnp.max(logits, axis=1, keepdims=True)
    p = jnp.exp(logits - m)
    scores = p / jnp.sum(p, axis=1, keepdims=True)
    col = jax.lax.broadcasted_iota(jnp.int32, scores.shape, 1)
    i1 = jnp.argmax(scores, axis=1)[:, None]
    w1 = jnp.max(scores, axis=1, keepdims=True)
    masked = jnp.where(col == i1, -jnp.inf, scores)
    i2 = jnp.argmax(masked, axis=1)[:, None]
    w2 = jnp.max(masked, axis=1, keepdims=True)
    wf_ref[...] = jnp.where(col == i1, w1, 0.0) + jnp.where(col == i2, w2, 0.0)


def _ffn_kernel(xb_ref, w1_ref, w3_ref, w2_ref, s_ref, q_ref):
    xb = xb_ref[...]
    dn = (((1,), (1,)), ((), ()))
    a = jax.lax.dot_general(xb, w1_ref[0], dn, preferred_element_type=jnp.float32)
    b = jax.lax.dot_general(xb, w3_ref[0], dn, preferred_element_type=jnp.float32)
    g = (a * jax.lax.logistic(a)) * b
    tau = jnp.sqrt(jnp.clip(jnp.sum(g * g, axis=1, keepdims=True) + C, 1e-6, None))
    col = jax.lax.broadcasted_iota(jnp.int32, g.shape, 1)
    xh = jnp.where(col == 0, tau, g).astype(jnp.bfloat16)
    s = jax.lax.dot_general(xh, w2_ref[0], dn, preferred_element_type=jnp.float32)
    s_ref[...] = s.astype(jnp.bfloat16)
    q_ref[...] = jnp.sum(s * s, axis=1, keepdims=True)


def _combine_kernel(wf_ref, scale_ref, *refs):
    s_refs = refs[:E + 1]
    q_refs = refs[E + 1:2 * (E + 1)]
    out_ref = refs[-1]
    wf = wf_ref[...]
    scale = scale_ref[0, 0]
    y_s = jnp.zeros((RB, DIM), jnp.float32)
    y_t = jnp.zeros((RB, 1), jnp.float32)
    for e in range(E):
        w = wf[:, e:e + 1]
        y_s = y_s + w * s_refs[e][...].astype(jnp.float32)
        y_t = y_t + w * jnp.sqrt(q_refs[e][...] + C)
    z_s = s_refs[E][...].astype(jnp.float32)
    z_t = jnp.sqrt(q_refs[E][...] + C)
    u_s = z_s + scale * y_s
    u_t = z_t + scale * y_t
    neg_inner = u_t * u_t - jnp.sum(u_s * u_s, axis=1, keepdims=True)
    denom = jnp.sqrt(jnp.clip(neg_inner, 1e-6, None) / C)
    col = jax.lax.broadcasted_iota(jnp.int32, u_s.shape, 1)
    out_ref[...] = jnp.where(col == 0, u_t, u_s) / denom


@functools.partial(jax.jit, static_argnums=())
def kernel(x, Wg, W1, W2, W3, Ws1, Ws2, Ws3, lres_scale):
    nb = TOK // RB
    ne = E + 1

    w1p = jnp.pad(jnp.concatenate([W1, Ws1[None]], 0), ((0, 0), (1, 0), (0, 0))).astype(jnp.bfloat16)
    w3p = jnp.pad(jnp.concatenate([W3, Ws3[None]], 0), ((0, 0), (1, 0), (0, 0))).astype(jnp.bfloat16)
    w2p = jnp.pad(jnp.concatenate([W2, Ws2[None]], 0), ((0, 0), (1, 0), (0, 0))).astype(jnp.bfloat16)
    xb = x.astype(jnp.bfloat16)

    wf = pl.pallas_call(
        _gate_kernel,
        grid=(nb,),
        in_specs=[
            pl.BlockSpec((RB, DIM), lambda i: (i, 0)),
            pl.BlockSpec((E, DIM), lambda i: (0, 0)),
        ],
        out_specs=pl.BlockSpec((RB, E), lambda i: (i, 0)),
        out_shape=jax.ShapeDtypeStruct((TOK, E), jnp.float32),
    )(x, Wg)

    s, q = pl.pallas_call(
        _ffn_kernel,
        grid=(ne * nb,),
        in_specs=[
            pl.BlockSpec((RB, DIM), lambda i: (i % nb, 0)),
            pl.BlockSpec((1, INTER, DIM), lambda i: (i // nb, 0, 0)),
            pl.BlockSpec((1, INTER, DIM), lambda i: (i // nb, 0, 0)),
            pl.BlockSpec((1, DIM, INTER), lambda i: (i // nb, 0, 0)),
        ],
        out_specs=[
            pl.BlockSpec((RB, DIM), lambda i: (i, 0)),
            pl.BlockSpec((RB, 1), lambda i: (i, 0)),
        ],
        out_shape=[
            jax.ShapeDtypeStruct((ne * TOK, DIM), jnp.bfloat16),
            jax.ShapeDtypeStruct((ne * TOK, 1), jnp.float32),
        ],
    )(xb, w1p, w3p, w2p)

    s_specs = [pl.BlockSpec((RB, DIM), functools.partial(lambda e, i: (e * (TOK // RB) + i, 0), e))
               for e in range(ne)]
    q_specs = [pl.BlockSpec((RB, 1), functools.partial(lambda e, i: (e * (TOK // RB) + i, 0), e))
               for e in range(ne)]
    out = pl.pallas_call(
        _combine_kernel,
        grid=(nb,),
        in_specs=[
            pl.BlockSpec((RB, E), lambda i: (i, 0)),
            pl.BlockSpec((1, 1), lambda i: (0, 0)),
        ] + s_specs + q_specs,
        out_specs=pl.BlockSpec((RB, DIM), lambda i: (i, 0)),
        out_shape=jax.ShapeDtypeStruct((TOK, DIM), jnp.float32),
    )(wf, lres_scale.reshape(1, 1), *([s] * ne), *([q] * ne))
    return out


# speedup vs baseline: 1.5726x; 1.3740x over previous
import functools

import jax
import jax.numpy as jnp
from jax.experimental import pallas as pl
from jax.experimental.pallas import tpu as pltpu

C = 1.0
DIM = 2048
INTER = 1408
E = 8
TOPK = 2
TOK = 2048
RB = 128


def _gate_kernel(x_ref, wg_ref, wf_ref):
    logits = jax.lax.dot_general(
        x_ref[...], wg_ref[...], (((1,), (1,)), ((), ())),
        preferred_element_type=jnp.float32,
        precision=jax.lax.Precision.DEFAULT)
    m = jnp.max(logits, axis=1, keepdims=True)
    p = jnp.exp(logits - m)
    scores = p / jnp.sum(p, axis=1, keepdims=True)
    col = jax.lax.broadcasted_iota(jnp.int32, scores.shape, 1)
    i1 = jnp.argmax(scores, axis=1)[:, None]
    w1 = jnp.max(scores, axis=1, keepdims=True)
    masked = jnp.where(col == i1, -jnp.inf, scores)
    i2 = jnp.argmax(masked, axis=1)[:, None]
    w2 = jnp.max(masked, axis=1, keepdims=True)
    wf_ref[...] = jnp.where(col == i1, w1, 0.0) + jnp.where(col == i2, w2, 0.0)


def _ffn_kernel(xb_ref, w1_ref, w3_ref, w2_ref, s_ref, q_ref):
    xb = xb_ref[...]
    dn = (((1,), (0,)), ((), ()))
    a = jax.lax.dot_general(xb, w1_ref[0], dn, preferred_element_type=jnp.float32)
    b = jax.lax.dot_general(xb, w3_ref[0], dn, preferred_element_type=jnp.float32)
    g = (a * jax.lax.logistic(a)) * b
    tau = jnp.sqrt(jnp.clip(jnp.sum(g * g, axis=1, keepdims=True) + C, 1e-6, None))
    col = jax.lax.broadcasted_iota(jnp.int32, g.shape, 1)
    xh = jnp.where(col == 0, tau, g).astype(jnp.bfloat16)
    s = jax.lax.dot_general(xh, w2_ref[0], dn, preferred_element_type=jnp.float32)
    s_ref[...] = s.astype(jnp.bfloat16)
    q_ref[...] = jnp.sum(s * s, axis=1, keepdims=True)


def _combine_kernel(wf_ref, scale_ref, *refs):
    s_refs = refs[:E + 1]
    q_refs = refs[E + 1:2 * (E + 1)]
    out_ref = refs[-1]
    wf = wf_ref[...]
    scale = scale_ref[0, 0]
    y_s = jnp.zeros((RB, DIM), jnp.float32)
    y_t = jnp.zeros((RB, 1), jnp.float32)
    for e in range(E):
        w = wf[:, e:e + 1]
        y_s = y_s + w * s_refs[e][...].astype(jnp.float32)
        y_t = y_t + w * jnp.sqrt(q_refs[e][...] + C)
    z_s = s_refs[E][...].astype(jnp.float32)
    z_t = jnp.sqrt(q_refs[E][...] + C)
    u_s = z_s + scale * y_s
    u_t = z_t + scale * y_t
    neg_inner = u_t * u_t - jnp.sum(u_s * u_s, axis=1, keepdims=True)
    denom = jnp.sqrt(jnp.clip(neg_inner, 1e-6, None) / C)
    col = jax.lax.broadcasted_iota(jnp.int32, u_s.shape, 1)
    out_ref[...] = jnp.where(col == 0, u_t, u_s) / denom


@functools.partial(jax.jit, static_argnums=())
def kernel(x, Wg, W1, W2, W3, Ws1, Ws2, Ws3, lres_scale):
    nb = TOK // RB
    ne = E + 1

    w1p = jnp.pad(jnp.concatenate([W1, Ws1[None]], 0).swapaxes(1, 2), ((0, 0), (0, 0), (1, 0))).astype(jnp.bfloat16)
    w3p = jnp.pad(jnp.concatenate([W3, Ws3[None]], 0).swapaxes(1, 2), ((0, 0), (0, 0), (1, 0))).astype(jnp.bfloat16)
    w2p = jnp.pad(jnp.concatenate([W2, Ws2[None]], 0).swapaxes(1, 2), ((0, 0), (0, 0), (1, 0))).astype(jnp.bfloat16)
    xb = x.astype(jnp.bfloat16)

    wf = pl.pallas_call(
        _gate_kernel,
        grid=(nb,),
        in_specs=[
            pl.BlockSpec((RB, DIM), lambda i: (i, 0)),
            pl.BlockSpec((E, DIM), lambda i: (0, 0)),
        ],
        out_specs=pl.BlockSpec((RB, E), lambda i: (i, 0)),
        out_shape=jax.ShapeDtypeStruct((TOK, E), jnp.float32),
    )(x, Wg)

    s, q = pl.pallas_call(
        _ffn_kernel,
        grid=(ne * nb,),
        in_specs=[
            pl.BlockSpec((RB, DIM), lambda i: (i % nb, 0)),
            pl.BlockSpec((1, DIM, INTER), lambda i: (i // nb, 0, 0)),
            pl.BlockSpec((1, DIM, INTER), lambda i: (i // nb, 0, 0)),
            pl.BlockSpec((1, INTER, DIM), lambda i: (i // nb, 0, 0)),
        ],
        out_specs=[
            pl.BlockSpec((RB, DIM), lambda i: (i, 0)),
            pl.BlockSpec((RB, 1), lambda i: (i, 0)),
        ],
        out_shape=[
            jax.ShapeDtypeStruct((ne * TOK, DIM), jnp.bfloat16),
            jax.ShapeDtypeStruct((ne * TOK, 1), jnp.float32),
        ],
    )(xb, w1p, w3p, w2p)

    s_specs = [pl.BlockSpec((RB, DIM), functools.partial(lambda e, i: (e * (TOK // RB) + i, 0), e))
               for e in range(ne)]
    q_specs = [pl.BlockSpec((RB, 1), functools.partial(lambda e, i: (e * (TOK // RB) + i, 0), e))
               for e in range(ne)]
    out = pl.pallas_call(
        _combine_kernel,
        grid=(nb,),
        in_specs=[
            pl.BlockSpec((RB, E), lambda i: (i, 0)),
            pl.BlockSpec((1, 1), lambda i: (0, 0)),
        ] + s_specs + q_specs,
        out_specs=pl.BlockSpec((RB, DIM), lambda i: (i, 0)),
        out_shape=jax.ShapeDtypeStruct((TOK, DIM), jnp.float32),
    )(wf, lres_scale.reshape(1, 1), *([s] * ne), *([q] * ne))
    return out


# speedup vs baseline: 1.7136x; 1.0897x over previous
import functools

import jax
import jax.numpy as jnp
from jax.experimental import pallas as pl
from jax.experimental.pallas import tpu as pltpu

C = 1.0
DIM = 2048
INTER = 1408
E = 8
TOPK = 2
TOK = 2048
RB = 256


def _gate_kernel(x_ref, wg_ref, wf_ref):
    logits = jax.lax.dot_general(
        x_ref[...], wg_ref[...], (((1,), (1,)), ((), ())),
        preferred_element_type=jnp.float32,
        precision=jax.lax.Precision.DEFAULT)
    m = jnp.max(logits, axis=1, keepdims=True)
    p = jnp.exp(logits - m)
    scores = p / jnp.sum(p, axis=1, keepdims=True)
    col = jax.lax.broadcasted_iota(jnp.int32, scores.shape, 1)
    i1 = jnp.argmax(scores, axis=1)[:, None]
    w1 = jnp.max(scores, axis=1, keepdims=True)
    masked = jnp.where(col == i1, -jnp.inf, scores)
    i2 = jnp.argmax(masked, axis=1)[:, None]
    w2 = jnp.max(masked, axis=1, keepdims=True)
    wf_ref[...] = jnp.where(col == i1, w1, 0.0) + jnp.where(col == i2, w2, 0.0)


def _ffn_kernel(xb_ref, w1_ref, w3_ref, w2_ref, s_ref, q_ref):
    xb = xb_ref[...]
    dn = (((1,), (0,)), ((), ()))
    a = jax.lax.dot_general(xb, w1_ref[0], dn, preferred_element_type=jnp.float32)
    b = jax.lax.dot_general(xb, w3_ref[0], dn, preferred_element_type=jnp.float32)
    g = (a * jax.lax.logistic(a)) * b
    tau = jnp.sqrt(jnp.clip(jnp.sum(g * g, axis=1, keepdims=True) + C, 1e-6, None))
    col = jax.lax.broadcasted_iota(jnp.int32, g.shape, 1)
    xh = jnp.where(col == 0, tau, g).astype(jnp.bfloat16)
    s = jax.lax.dot_general(xh, w2_ref[0], dn, preferred_element_type=jnp.float32)
    s_ref[...] = s.astype(jnp.bfloat16)
    q_ref[...] = jnp.sum(s * s, axis=1, keepdims=True)


def _combine_kernel(wf_ref, scale_ref, *refs):
    s_refs = refs[:E + 1]
    q_refs = refs[E + 1:2 * (E + 1)]
    out_ref = refs[-1]
    wf = wf_ref[...]
    scale = scale_ref[0, 0]
    y_s = jnp.zeros((RB, DIM), jnp.float32)
    y_t = jnp.zeros((RB, 1), jnp.float32)
    for e in range(E):
        w = wf[:, e:e + 1]
        y_s = y_s + w * s_refs[e][...].astype(jnp.float32)
        y_t = y_t + w * jnp.sqrt(q_refs[e][...] + C)
    z_s = s_refs[E][...].astype(jnp.float32)
    z_t = jnp.sqrt(q_refs[E][...] + C)
    u_s = z_s + scale * y_s
    u_t = z_t + scale * y_t
    neg_inner = u_t * u_t - jnp.sum(u_s * u_s, axis=1, keepdims=True)
    denom = jnp.sqrt(jnp.clip(neg_inner, 1e-6, None) / C)
    col = jax.lax.broadcasted_iota(jnp.int32, u_s.shape, 1)
    out_ref[...] = jnp.where(col == 0, u_t, u_s) / denom


@functools.partial(jax.jit, static_argnums=())
def kernel(x, Wg, W1, W2, W3, Ws1, Ws2, Ws3, lres_scale):
    nb = TOK // RB
    ne = E + 1

    w1p = jnp.pad(jnp.concatenate([W1, Ws1[None]], 0).swapaxes(1, 2), ((0, 0), (0, 0), (1, 0))).astype(jnp.bfloat16)
    w3p = jnp.pad(jnp.concatenate([W3, Ws3[None]], 0).swapaxes(1, 2), ((0, 0), (0, 0), (1, 0))).astype(jnp.bfloat16)
    w2p = jnp.pad(jnp.concatenate([W2, Ws2[None]], 0).swapaxes(1, 2), ((0, 0), (0, 0), (1, 0))).astype(jnp.bfloat16)
    xb = x.astype(jnp.bfloat16)

    wf = pl.pallas_call(
        _gate_kernel,
        grid=(nb,),
        in_specs=[
            pl.BlockSpec((RB, DIM), lambda i: (i, 0)),
            pl.BlockSpec((E, DIM), lambda i: (0, 0)),
        ],
        out_specs=pl.BlockSpec((RB, E), lambda i: (i, 0)),
        out_shape=jax.ShapeDtypeStruct((TOK, E), jnp.float32),
    )(x, Wg)

    s, q = pl.pallas_call(
        _ffn_kernel,
        grid=(ne * nb,),
        in_specs=[
            pl.BlockSpec((RB, DIM), lambda i: (i % nb, 0)),
            pl.BlockSpec((1, DIM, INTER), lambda i: (i // nb, 0, 0)),
            pl.BlockSpec((1, DIM, INTER), lambda i: (i // nb, 0, 0)),
            pl.BlockSpec((1, INTER, DIM), lambda i: (i // nb, 0, 0)),
        ],
        out_specs=[
            pl.BlockSpec((RB, DIM), lambda i: (i, 0)),
            pl.BlockSpec((RB, 1), lambda i: (i, 0)),
        ],
        out_shape=[
            jax.ShapeDtypeStruct((ne * TOK, DIM), jnp.bfloat16),
            jax.ShapeDtypeStruct((ne * TOK, 1), jnp.float32),
        ],
    )(xb, w1p, w3p, w2p)

    s_specs = [pl.BlockSpec((RB, DIM), functools.partial(lambda e, i: (e * (TOK // RB) + i, 0), e))
               for e in range(ne)]
    q_specs = [pl.BlockSpec((RB, 1), functools.partial(lambda e, i: (e * (TOK // RB) + i, 0), e))
               for e in range(ne)]
    out = pl.pallas_call(
        _combine_kernel,
        grid=(nb,),
        in_specs=[
            pl.BlockSpec((RB, E), lambda i: (i, 0)),
            pl.BlockSpec((1, 1), lambda i: (0, 0)),
        ] + s_specs + q_specs,
        out_specs=pl.BlockSpec((RB, DIM), lambda i: (i, 0)),
        out_shape=jax.ShapeDtypeStruct((TOK, DIM), jnp.float32),
    )(wf, lres_scale.reshape(1, 1), *([s] * ne), *([q] * ne))
    return out
